# baseline (speedup 1.0000x reference)
import functools

import jax
import jax.numpy as jnp
from jax import lax
from jax.experimental import pallas as pl
from jax.experimental.pallas import tpu as pltpu
from jax.experimental.pallas import tpu_sc as plsc

E = 64
NG = 8
GSZ = E // NG
TOPKG = 3
K = 8
BLOCK_T = 2048

NC = 2
NS = 16
NW = NC * NS
L = 16


def _score_kernel(x_ref, wt_ref, s_ref):
    logits = jnp.dot(x_ref[...], wt_ref[...],
                     preferred_element_type=jnp.float32)
    m = jnp.max(logits, axis=1, keepdims=True)
    unnorm = jnp.exp(logits - m)
    s_ref[...] = unnorm / jnp.sum(unnorm, axis=1, keepdims=True)


def _perm(x, idx):
    return lax.gather(
        x, idx[:, None],
        dimension_numbers=lax.GatherDimensionNumbers(
            offset_dims=(), collapsed_slice_dims=(0,), start_index_map=(0,)),
        slice_sizes=(1,),
        mode=lax.GatherScatterMode.PROMISE_IN_BOUNDS)


def _make_route_kernel(n_tok):
    tpw = n_tok // NW
    mesh = plsc.VectorSubcoreMesh(core_axis_name="c", subcore_axis_name="s")

    @functools.partial(
        pl.kernel, mesh=mesh,
        out_type=[
            jax.ShapeDtypeStruct((n_tok * K,), jnp.int32),
            jax.ShapeDtypeStruct((n_tok * K,), jnp.float32),
        ],
        scratch_types=[
            pltpu.VMEM((tpw, E), jnp.float32),
            pltpu.VMEM((tpw * K + K,), jnp.int32),
            pltpu.VMEM((tpw * K + K,), jnp.float32),
        ],
        compiler_params=pltpu.CompilerParams(needs_layout_passes=False),
    )
    def route(scores_hbm, idx_hbm, wgt_hbm, sbuf, ibuf, wbuf):
        wid = lax.axis_index("s") * NC + lax.axis_index("c")
        base = wid * tpw
        pltpu.sync_copy(scores_hbm.at[pl.ds(base, tpw), :], sbuf)

        lane = lax.broadcasted_iota(jnp.int32, (L,), 0)
        half = lane & 7
        pick08 = (lane & 1) * 8

        def body(t, carry):
            s = [sbuf[t, pl.ds(16 * v, 16)] for v in range(4)]
            gm = []
            for v in range(4):
                m = s[v]
                for sh in (4, 2, 1):
                    m = jnp.maximum(m, _perm(m, lane ^ sh))
                gm.append(m)
            G = jnp.full((L,), -1.0, jnp.float32)
            for v in range(4):
                sel = (lane >> 1) == v
                G = jnp.where(sel, _perm(gm[v], pick08), G)
            G = jnp.where(lane < NG, G, -1.0)
            _, sg = plsc.sort_key_val(G, lane, descending=True)
            b = [_perm(sg, jnp.full((L,), i, jnp.int32)) for i in range(TOPKG)]
            ks, vs = [], []
            for v in range(4):
                gid = 2 * v + (lane >> 3)
                keep = (gid == b[0]) | (gid == b[1]) | (gid == b[2])
                ms = jnp.where(keep, s[v], 0.0)
                kk, vv = plsc.sort_key_val(ms, 16 * v + lane, descending=True)
                ks.append(kk)
                vs.append(vv)

            def merge(ka, ia, kb, ib):
                ck = jnp.where(lane < 8, ka, _perm(kb, half))
                ci = jnp.where(lane < 8, ia, _perm(ib, half))
                return plsc.sort_key_val(ck, ci, descending=True)

            ka, ia = merge(ks[0], vs[0], ks[1], vs[1])
            kb, ib = merge(ks[2], vs[2], ks[3], vs[3])
            kf, idxf = merge(ka, ia, kb, ib)

            top = lane < K
            total = jnp.sum(jnp.where(top, kf, 0.0))
            wv = kf / (total + 1e-20)
            plsc.store_compressed(ibuf.at[pl.ds(t * K, L)], idxf, mask=top)
            plsc.store_compressed(wbuf.at[pl.ds(t * K, L)], wv, mask=top)
            return carry

        lax.fori_loop(0, tpw, body, 0)
        pltpu.sync_copy(ibuf.at[pl.ds(0, tpw * K)],
                        idx_hbm.at[pl.ds(base * K, tpw * K)])
        pltpu.sync_copy(wbuf.at[pl.ds(0, tpw * K)],
                        wgt_hbm.at[pl.ds(base * K, tpw * K)])

    return route


@jax.jit
def kernel(hidden_states, weight):
    bsz, seq, h = hidden_states.shape
    x = hidden_states.reshape(-1, h)
    n_tok = x.shape[0]
    scores = pl.pallas_call(
        _score_kernel,
        grid=(n_tok // BLOCK_T,),
        in_specs=[
            pl.BlockSpec((BLOCK_T, h), lambda i: (i, 0)),
            pl.BlockSpec((h, E), lambda i: (0, 0)),
        ],
        out_specs=pl.BlockSpec((BLOCK_T, E), lambda i: (i, 0)),
        out_shape=jax.ShapeDtypeStruct((n_tok, E), jnp.float32),
        compiler_params=pltpu.CompilerParams(
            dimension_semantics=("arbitrary",),
        ),
    )(x, weight.T)
    idx_flat, wgt_flat = _make_route_kernel(n_tok)(scores)
    return (idx_flat.reshape(n_tok, K), wgt_flat.reshape(n_tok, K), None)


# speedup vs baseline: 1.2663x; 1.2663x over previous
import functools

import jax
import jax.numpy as jnp
from jax import lax
from jax.experimental import pallas as pl
from jax.experimental.pallas import tpu as pltpu
from jax.experimental.pallas import tpu_sc as plsc

E = 64
NG = 8
GSZ = E // NG
TOPKG = 3
K = 8
BLOCK_T = 2048

NC = 2
NS = 16
NW = NC * NS
L = 16


def _score_kernel(x_ref, wt_ref, s_ref):
    logits = jnp.dot(x_ref[...], wt_ref[...],
                     preferred_element_type=jnp.float32)
    m = jnp.max(logits, axis=1, keepdims=True)
    unnorm = jnp.exp(logits - m)
    s_ref[...] = unnorm / jnp.sum(unnorm, axis=1, keepdims=True)


def _perm(x, idx):
    return lax.gather(
        x, idx[:, None],
        dimension_numbers=lax.GatherDimensionNumbers(
            offset_dims=(), collapsed_slice_dims=(0,), start_index_map=(0,)),
        slice_sizes=(1,),
        mode=lax.GatherScatterMode.PROMISE_IN_BOUNDS)


def _make_route_kernel(n_tok):
    tpw = n_tok // NW
    mesh = plsc.VectorSubcoreMesh(core_axis_name="c", subcore_axis_name="s")

    @functools.partial(
        pl.kernel, mesh=mesh,
        out_type=[
            jax.ShapeDtypeStruct((n_tok * K,), jnp.int32),
            jax.ShapeDtypeStruct((n_tok * K,), jnp.float32),
        ],
        scratch_types=[
            pltpu.VMEM((tpw, E), jnp.float32),
            pltpu.VMEM((tpw * K + K,), jnp.int32),
            pltpu.VMEM((tpw * K + K,), jnp.float32),
        ],
        compiler_params=pltpu.CompilerParams(needs_layout_passes=False),
    )
    def route(scores_hbm, idx_hbm, wgt_hbm, sbuf, ibuf, wbuf):
        wid = lax.axis_index("s") * NC + lax.axis_index("c")
        base = wid * tpw
        pltpu.sync_copy(scores_hbm.at[pl.ds(base, tpw), :], sbuf)

        lane = lax.broadcasted_iota(jnp.int32, (L,), 0)
        half = lane & 7
        pick08 = (lane & 1) * 8

        @plsc.parallel_loop(0, tpw, step=1, unroll=4)
        def body(t):
            s = [sbuf[t, pl.ds(16 * v, 16)] for v in range(4)]
            gm = []
            for v in range(4):
                m = s[v]
                for sh in (4, 2, 1):
                    m = jnp.maximum(m, _perm(m, lane ^ sh))
                gm.append(m)
            G = jnp.full((L,), -1.0, jnp.float32)
            for v in range(4):
                sel = (lane >> 1) == v
                G = jnp.where(sel, _perm(gm[v], pick08), G)
            G = jnp.where(lane < NG, G, -1.0)
            _, sg = plsc.sort_key_val(G, lane, descending=True)
            b = [_perm(sg, jnp.full((L,), i, jnp.int32)) for i in range(TOPKG)]
            ks, vs = [], []
            for v in range(4):
                gid = 2 * v + (lane >> 3)
                keep = (gid == b[0]) | (gid == b[1]) | (gid == b[2])
                ms = jnp.where(keep, s[v], 0.0)
                kk, vv = plsc.sort_key_val(ms, 16 * v + lane, descending=True)
                ks.append(kk)
                vs.append(vv)

            def merge(ka, ia, kb, ib):
                ck = jnp.where(lane < 8, ka, _perm(kb, half))
                ci = jnp.where(lane < 8, ia, _perm(ib, half))
                return plsc.sort_key_val(ck, ci, descending=True)

            ka, ia = merge(ks[0], vs[0], ks[1], vs[1])
            kb, ib = merge(ks[2], vs[2], ks[3], vs[3])
            kf, idxf = merge(ka, ia, kb, ib)

            top = lane < K
            total = jnp.sum(jnp.where(top, kf, 0.0))
            wv = kf / (total + 1e-20)
            plsc.store_compressed(ibuf.at[pl.ds(t * K, L)], idxf, mask=top)
            plsc.store_compressed(wbuf.at[pl.ds(t * K, L)], wv, mask=top)
        pltpu.sync_copy(ibuf.at[pl.ds(0, tpw * K)],
                        idx_hbm.at[pl.ds(base * K, tpw * K)])
        pltpu.sync_copy(wbuf.at[pl.ds(0, tpw * K)],
                        wgt_hbm.at[pl.ds(base * K, tpw * K)])

    return route


@jax.jit
def kernel(hidden_states, weight):
    bsz, seq, h = hidden_states.shape
    x = hidden_states.reshape(-1, h)
    n_tok = x.shape[0]
    scores = pl.pallas_call(
        _score_kernel,
        grid=(n_tok // BLOCK_T,),
        in_specs=[
            pl.BlockSpec((BLOCK_T, h), lambda i: (i, 0)),
            pl.BlockSpec((h, E), lambda i: (0, 0)),
        ],
        out_specs=pl.BlockSpec((BLOCK_T, E), lambda i: (i, 0)),
        out_shape=jax.ShapeDtypeStruct((n_tok, E), jnp.float32),
        compiler_params=pltpu.CompilerParams(
            dimension_semantics=("arbitrary",),
        ),
    )(x, weight.T)
    idx_flat, wgt_flat = _make_route_kernel(n_tok)(scores)
    return (idx_flat.reshape(n_tok, K), wgt_flat.reshape(n_tok, K), None)


# speedup vs baseline: 1.2868x; 1.0162x over previous
import functools

import jax
import jax.numpy as jnp
from jax import lax
from jax.experimental import pallas as pl
from jax.experimental.pallas import tpu as pltpu
from jax.experimental.pallas import tpu_sc as plsc

E = 64
NG = 8
GSZ = E // NG
TOPKG = 3
K = 8
BLOCK_T = 2048

NC = 2
NS = 16
NW = NC * NS
L = 16


def _score_kernel(x_ref, wt_ref, s_ref):
    logits = jnp.dot(x_ref[...], wt_ref[...],
                     preferred_element_type=jnp.float32)
    m = jnp.max(logits, axis=1, keepdims=True)
    unnorm = jnp.exp(logits - m)
    s_ref[...] = unnorm / jnp.sum(unnorm, axis=1, keepdims=True)


def _perm(x, idx):
    return lax.gather(
        x, idx[:, None],
        dimension_numbers=lax.GatherDimensionNumbers(
            offset_dims=(), collapsed_slice_dims=(0,), start_index_map=(0,)),
        slice_sizes=(1,),
        mode=lax.GatherScatterMode.PROMISE_IN_BOUNDS)


def _make_route_kernel(n_tok):
    tpw = n_tok // NW
    mesh = plsc.VectorSubcoreMesh(core_axis_name="c", subcore_axis_name="s")

    @functools.partial(
        pl.kernel, mesh=mesh,
        out_type=[
            jax.ShapeDtypeStruct((n_tok * K,), jnp.int32),
            jax.ShapeDtypeStruct((n_tok * K,), jnp.float32),
        ],
        scratch_types=[
            pltpu.VMEM((tpw, E), jnp.float32),
            pltpu.VMEM((tpw * K + K,), jnp.int32),
            pltpu.VMEM((tpw * K + K,), jnp.float32),
        ],
        compiler_params=pltpu.CompilerParams(needs_layout_passes=False),
    )
    def route(scores_hbm, idx_hbm, wgt_hbm, sbuf, ibuf, wbuf):
        wid = lax.axis_index("s") * NC + lax.axis_index("c")
        base = wid * tpw
        pltpu.sync_copy(scores_hbm.at[pl.ds(base, tpw), :], sbuf)

        lane = lax.broadcasted_iota(jnp.int32, (L,), 0)
        half = lane & 7

        thr_idx = jnp.full((L,), TOPKG - 1, jnp.int32)
        src0 = lane & 3
        gpack = (lane & 7) < 4

        @plsc.parallel_loop(0, tpw, step=1, unroll=8)
        def body(t):
            s = [sbuf[t, pl.ds(16 * v, 16)] for v in range(4)]
            gm = []
            for v in range(4):
                m = s[v]
                for sh in (4, 2, 1):
                    m = jnp.maximum(m, _perm(m, lane ^ sh))
                gm.append(m)
            G = jnp.where(src0 == 0, gm[0],
                jnp.where(src0 == 1, gm[1],
                jnp.where(src0 == 2, gm[2], gm[3])))
            G = jnp.where(gpack, G, -1.0)
            sk, _ = plsc.sort_key_val(G, lane, descending=True)
            thr = _perm(sk, thr_idx)
            ks, vs = [], []
            for v in range(4):
                ms = jnp.where(gm[v] >= thr, s[v], 0.0)
                kk, vv = plsc.sort_key_val(ms, 16 * v + lane, descending=True)
                ks.append(kk)
                vs.append(vv)

            def merge(ka, ia, kb, ib):
                ck = jnp.where(lane < 8, ka, _perm(kb, half))
                ci = jnp.where(lane < 8, ia, _perm(ib, half))
                return plsc.sort_key_val(ck, ci, descending=True)

            ka, ia = merge(ks[0], vs[0], ks[1], vs[1])
            kb, ib = merge(ks[2], vs[2], ks[3], vs[3])
            kf, idxf = merge(ka, ia, kb, ib)

            top = lane < K
            total = jnp.sum(jnp.where(top, kf, 0.0))
            wv = kf / (total + 1e-20)
            plsc.store_compressed(ibuf.at[pl.ds(t * K, L)], idxf, mask=top)
            plsc.store_compressed(wbuf.at[pl.ds(t * K, L)], wv, mask=top)
        pltpu.sync_copy(ibuf.at[pl.ds(0, tpw * K)],
                        idx_hbm.at[pl.ds(base * K, tpw * K)])
        pltpu.sync_copy(wbuf.at[pl.ds(0, tpw * K)],
                        wgt_hbm.at[pl.ds(base * K, tpw * K)])

    return route


NCHUNK = 1


@jax.jit
def kernel(hidden_states, weight):
    bsz, seq, h = hidden_states.shape
    x = hidden_states.reshape(-1, h)
    n_tok = x.shape[0]
    csz = n_tok // NCHUNK
    wt = weight.T
    route = _make_route_kernel(csz)
    idx_parts, wgt_parts = [], []
    for c in range(NCHUNK):
        scores = pl.pallas_call(
            _score_kernel,
            grid=(csz // BLOCK_T,),
            in_specs=[
                pl.BlockSpec((BLOCK_T, h), lambda i: (i, 0)),
                pl.BlockSpec((h, E), lambda i: (0, 0)),
            ],
            out_specs=pl.BlockSpec((BLOCK_T, E), lambda i: (i, 0)),
            out_shape=jax.ShapeDtypeStruct((csz, E), jnp.float32),
            compiler_params=pltpu.CompilerParams(
                dimension_semantics=("arbitrary",),
            ),
        )(lax.dynamic_slice_in_dim(x, c * csz, csz, 0), wt)
        idx_flat, wgt_flat = route(scores)
        idx_parts.append(idx_flat.reshape(csz, K))
        wgt_parts.append(wgt_flat.reshape(csz, K))
    return (jnp.concatenate(idx_parts, axis=0),
            jnp.concatenate(wgt_parts, axis=0), None)


# speedup vs baseline: 1.2932x; 1.0049x over previous
import functools

import jax
import jax.numpy as jnp
from jax import lax
from jax.experimental import pallas as pl
from jax.experimental.pallas import tpu as pltpu
from jax.experimental.pallas import tpu_sc as plsc

E = 64
NG = 8
GSZ = E // NG
TOPKG = 3
K = 8
BLOCK_T = 2048

NC = 2
NS = 16
NW = NC * NS
L = 16


def _score_kernel(x_ref, wt_ref, s_ref):
    logits = jnp.dot(x_ref[...], wt_ref[...],
                     preferred_element_type=jnp.float32)
    m = jnp.max(logits, axis=1, keepdims=True)
    unnorm = jnp.exp(logits - m)
    s_ref[...] = unnorm / jnp.sum(unnorm, axis=1, keepdims=True)


def _perm(x, idx):
    return lax.gather(
        x, idx[:, None],
        dimension_numbers=lax.GatherDimensionNumbers(
            offset_dims=(), collapsed_slice_dims=(0,), start_index_map=(0,)),
        slice_sizes=(1,),
        mode=lax.GatherScatterMode.PROMISE_IN_BOUNDS)


def _make_route_kernel(n_tok):
    tpw = n_tok // NW
    mesh = plsc.VectorSubcoreMesh(core_axis_name="c", subcore_axis_name="s")

    @functools.partial(
        pl.kernel, mesh=mesh,
        out_type=[
            jax.ShapeDtypeStruct((n_tok * K,), jnp.int32),
            jax.ShapeDtypeStruct((n_tok * K,), jnp.float32),
        ],
        scratch_types=[
            pltpu.VMEM((tpw, E), jnp.float32),
            pltpu.VMEM((tpw * K + K,), jnp.int32),
            pltpu.VMEM((tpw * K + K,), jnp.float32),
        ],
        compiler_params=pltpu.CompilerParams(needs_layout_passes=False),
    )
    def route(scores_hbm, idx_hbm, wgt_hbm, sbuf, ibuf, wbuf):
        wid = lax.axis_index("s") * NC + lax.axis_index("c")
        base = wid * tpw
        pltpu.sync_copy(scores_hbm.at[pl.ds(base, tpw), :], sbuf)

        lane = lax.broadcasted_iota(jnp.int32, (L,), 0)

        thr_idx = jnp.full((L,), TOPKG - 1, jnp.int32)
        src0 = lane & 3
        gpack = (lane & 7) < 4

        @plsc.parallel_loop(0, tpw, step=1, unroll=8)
        def body(t):
            s = [sbuf[t, pl.ds(16 * v, 16)] for v in range(4)]
            gm = []
            for v in range(4):
                m = s[v]
                for sh in (4, 2, 1):
                    m = jnp.maximum(m, _perm(m, lane ^ sh))
                gm.append(m)
            G = jnp.where(src0 == 0, gm[0],
                jnp.where(src0 == 1, gm[1],
                jnp.where(src0 == 2, gm[2], gm[3])))
            G = jnp.where(gpack, G, -1.0)
            sk, _ = plsc.sort_key_val(G, lane, descending=True)
            thr = _perm(sk, thr_idx)
            ks, vs = [], []
            for v in range(4):
                ms = jnp.where(gm[v] >= thr, s[v], 0.0)
                kk, vv = plsc.sort_key_val(ms, 16 * v + lane,
                                           descending=(v % 2 == 0))
                ks.append(kk)
                vs.append(vv)

            def merge(ka, ia, kb, ib, descending):
                ck = jnp.where(lane < 8, ka, kb)
                ci = jnp.where(lane < 8, ia, ib)
                return plsc.sort_key_val(ck, ci, descending=descending)

            ka, ia = merge(ks[0], vs[0], ks[1], vs[1], True)
            kb, ib = merge(ks[2], vs[2], ks[3], vs[3], False)
            kf, idxf = merge(ka, ia, kb, ib, True)

            top = lane < K
            total = jnp.sum(jnp.where(top, kf, 0.0))
            wv = kf / (total + 1e-20)
            plsc.store_compressed(ibuf.at[pl.ds(t * K, L)], idxf, mask=top)
            plsc.store_compressed(wbuf.at[pl.ds(t * K, L)], wv, mask=top)
        pltpu.sync_copy(ibuf.at[pl.ds(0, tpw * K)],
                        idx_hbm.at[pl.ds(base * K, tpw * K)])
        pltpu.sync_copy(wbuf.at[pl.ds(0, tpw * K)],
                        wgt_hbm.at[pl.ds(base * K, tpw * K)])

    return route


NCHUNK = 1


@jax.jit
def kernel(hidden_states, weight):
    bsz, seq, h = hidden_states.shape
    x = hidden_states.reshape(-1, h)
    n_tok = x.shape[0]
    csz = n_tok // NCHUNK
    wt = weight.T
    route = _make_route_kernel(csz)
    idx_parts, wgt_parts = [], []
    for c in range(NCHUNK):
        scores = pl.pallas_call(
            _score_kernel,
            grid=(csz // BLOCK_T,),
            in_specs=[
                pl.BlockSpec((BLOCK_T, h), lambda i: (i, 0)),
                pl.BlockSpec((h, E), lambda i: (0, 0)),
            ],
            out_specs=pl.BlockSpec((BLOCK_T, E), lambda i: (i, 0)),
            out_shape=jax.ShapeDtypeStruct((csz, E), jnp.float32),
            compiler_params=pltpu.CompilerParams(
                dimension_semantics=("arbitrary",),
            ),
        )(lax.dynamic_slice_in_dim(x, c * csz, csz, 0), wt)
        idx_flat, wgt_flat = route(scores)
        idx_parts.append(idx_flat.reshape(csz, K))
        wgt_parts.append(wgt_flat.reshape(csz, K))
    return (jnp.concatenate(idx_parts, axis=0),
            jnp.concatenate(wgt_parts, axis=0), None)


# speedup vs baseline: 1.9285x; 1.4913x over previous
import functools

import jax
import jax.numpy as jnp
from jax import lax
from jax.experimental import pallas as pl
from jax.experimental.pallas import tpu as pltpu
from jax.experimental.pallas import tpu_sc as plsc

E = 64
NG = 8
GSZ = E // NG
TOPKG = 3
K = 8
BLOCK_T = 2048

NC = 2
NS = 16
NW = NC * NS
L = 16

_SORT8 = [(0, 1), (2, 3), (4, 5), (6, 7),
          (0, 2), (1, 3), (4, 6), (5, 7),
          (1, 2), (5, 6),
          (0, 4), (1, 5), (2, 6), (3, 7),
          (2, 4), (3, 5),
          (1, 2), (3, 4), (5, 6)]
_BITONIC8 = [(0, 4), (1, 5), (2, 6), (3, 7),
             (0, 2), (1, 3), (4, 6), (5, 7),
             (0, 1), (2, 3), (4, 5), (6, 7)]


def _score_kernel(x_ref, w_ref, s_ref):
    logits = lax.dot_general(
        w_ref[...], x_ref[...],
        dimension_numbers=(((1,), (1,)), ((), ())),
        preferred_element_type=jnp.float32,
    )
    m = jnp.max(logits, axis=0, keepdims=True)
    unnorm = jnp.exp(logits - m)
    s_ref[...] = unnorm / jnp.sum(unnorm, axis=0, keepdims=True)


def _cmpx(k, i, a, b):
    c = k[a] >= k[b]
    ka, kb = jnp.where(c, k[a], k[b]), jnp.where(c, k[b], k[a])
    ia, ib = jnp.where(c, i[a], i[b]), jnp.where(c, i[b], i[a])
    k[a], k[b], i[a], i[b] = ka, kb, ia, ib


def _sort8(k, i):
    for a, b in _SORT8:
        _cmpx(k, i, a, b)


def _merge8(ak, ai, bk, bi):
    wk, wi = [], []
    for j in range(8):
        c = ak[j] >= bk[7 - j]
        wk.append(jnp.where(c, ak[j], bk[7 - j]))
        wi.append(jnp.where(c, ai[j], bi[7 - j]))
    for a, b in _BITONIC8:
        _cmpx(wk, wi, a, b)
    return wk, wi


def _make_route_kernel(n_tok):
    tpw = n_tok // NW
    ntile = tpw // L
    mesh = plsc.VectorSubcoreMesh(core_axis_name="c", subcore_axis_name="s")

    @functools.partial(
        pl.kernel, mesh=mesh,
        out_type=[
            jax.ShapeDtypeStruct((K, n_tok), jnp.int32),
            jax.ShapeDtypeStruct((K, n_tok), jnp.float32),
        ],
        scratch_types=[
            pltpu.VMEM((E, tpw), jnp.float32),
            pltpu.VMEM((K, tpw), jnp.int32),
            pltpu.VMEM((K, tpw), jnp.float32),
        ],
        compiler_params=pltpu.CompilerParams(needs_layout_passes=False),
    )
    def route(scores_hbm, idx_hbm, wgt_hbm, sbuf, ibuf, wbuf):
        wid = lax.axis_index("s") * NC + lax.axis_index("c")
        base = wid * tpw
        pltpu.sync_copy(scores_hbm.at[:, pl.ds(base, tpw)], sbuf)

        lane = lax.broadcasted_iota(jnp.int32, (L,), 0)
        gids = [jnp.full((L,), g, jnp.int32) for g in range(NG)]

        @plsc.parallel_loop(0, ntile, step=1, unroll=2)
        def body(tile):
            tok = tile * L + lane
            gk = []
            for g in range(NG):
                m = sbuf[g * GSZ, pl.ds(tile * L, L)]
                for o in range(1, GSZ):
                    m = jnp.maximum(m, sbuf[g * GSZ + o, pl.ds(tile * L, L)])
                gk.append(m)
            gi = list(gids)
            _sort8(gk, gi)
            ck, ci = [], []
            for slot in range(TOPKG):
                erow0 = gi[slot] * GSZ
                for o in range(GSZ):
                    eid = erow0 + o
                    ck.append(plsc.load_gather(sbuf, [eid, tok]))
                    ci.append(eid)
            k0, i0 = ck[0:8], ci[0:8]
            k1, i1 = ck[8:16], ci[8:16]
            k2, i2 = ck[16:24], ci[16:24]
            _sort8(k0, i0)
            _sort8(k1, i1)
            _sort8(k2, i2)
            mk, mi = _merge8(k0, i0, k1, i1)
            fk, fi = _merge8(mk, mi, k2, i2)
            total = fk[0]
            for r in range(1, K):
                total = total + fk[r]
            total = total + 1e-20
            for r in range(K):
                ibuf[r, pl.ds(tile * L, L)] = fi[r]
                wbuf[r, pl.ds(tile * L, L)] = fk[r] / total
        pltpu.sync_copy(ibuf, idx_hbm.at[:, pl.ds(base, tpw)])
        pltpu.sync_copy(wbuf, wgt_hbm.at[:, pl.ds(base, tpw)])

    return route


@jax.jit
def kernel(hidden_states, weight):
    bsz, seq, h = hidden_states.shape
    x = hidden_states.reshape(-1, h)
    n_tok = x.shape[0]
    scores = pl.pallas_call(
        _score_kernel,
        grid=(n_tok // BLOCK_T,),
        in_specs=[
            pl.BlockSpec((BLOCK_T, h), lambda i: (i, 0)),
            pl.BlockSpec((E, h), lambda i: (0, 0)),
        ],
        out_specs=pl.BlockSpec((E, BLOCK_T), lambda i: (0, i)),
        out_shape=jax.ShapeDtypeStruct((E, n_tok), jnp.float32),
        compiler_params=pltpu.CompilerParams(
            dimension_semantics=("arbitrary",),
        ),
    )(x, weight)
    idx_t, wgt_t = _make_route_kernel(n_tok)(scores)
    return idx_t.T, wgt_t.T, None
